# baseline (device time: 132135 ns/iter reference)
import jax
import jax.numpy as jnp
from jax import lax
from jax.experimental import pallas as pl
from jax.experimental.pallas import tpu as pltpu

N_DEV = 16


def kernel(A, B):
    m, k_per = A.shape
    _, n = B.shape
    m_per = m // N_DEV

    def body(a_ref, b_ref, out_ref, comm_ref, send_sems, recv_sems):
        my = lax.axis_index("i")
        left = (my + N_DEV - 1) % N_DEV
        right = (my + 1) % N_DEV

        barrier_sem = pltpu.get_barrier_semaphore()
        for nbr in [left, right]:
            pl.semaphore_signal(
                barrier_sem, inc=1,
                device_id=(nbr,), device_id_type=pl.DeviceIdType.MESH,
            )

        c0 = (my + N_DEV - 1) % N_DEV
        comm_ref[0] = jnp.dot(
            a_ref[pl.ds(c0 * m_per, m_per), :], b_ref[...],
            preferred_element_type=jnp.float32,
        )

        pl.semaphore_wait(barrier_sem, 2)

        for s in range(N_DEV - 1):
            rdma = pltpu.make_async_remote_copy(
                src_ref=comm_ref.at[s],
                dst_ref=comm_ref.at[s + 1],
                send_sem=send_sems.at[s],
                recv_sem=recv_sems.at[s + 1],
                device_id=(right,),
                device_id_type=pl.DeviceIdType.MESH,
            )
            rdma.start()
            c = (my + 2 * N_DEV - 2 - s) % N_DEV
            partial = jnp.dot(
                a_ref[pl.ds(c * m_per, m_per), :], b_ref[...],
                preferred_element_type=jnp.float32,
            )
            rdma.wait()
            if s < N_DEV - 2:
                comm_ref[s + 1] = comm_ref[s + 1] + partial
            else:
                out_ref[...] = comm_ref[s + 1] + partial

    return pl.pallas_call(
        body,
        out_shape=jax.ShapeDtypeStruct((m_per, n), jnp.float32),
        in_specs=[
            pl.BlockSpec(memory_space=pltpu.VMEM),
            pl.BlockSpec(memory_space=pltpu.VMEM),
        ],
        out_specs=pl.BlockSpec(memory_space=pltpu.VMEM),
        scratch_shapes=[
            pltpu.VMEM((N_DEV, m_per, n), jnp.float32),
            pltpu.SemaphoreType.DMA((N_DEV,)),
            pltpu.SemaphoreType.DMA((N_DEV,)),
        ],
        compiler_params=pltpu.CompilerParams(collective_id=0),
    )(A, B)


# device time: 83822 ns/iter; 1.5764x vs baseline; 1.5764x over previous
import jax
import jax.numpy as jnp
from jax import lax
from jax.experimental import pallas as pl
from jax.experimental.pallas import tpu as pltpu

N_DEV = 16

P = [0, 1, 5, 4, 8, 9, 13, 12, 15, 14, 10, 11, 7, 6, 2, 3]
INV = [0] * N_DEV
for _r, _l in enumerate(P):
    INV[_l] = _r


def kernel(A, B):
    m, k_per = A.shape
    _, n = B.shape
    m_per = m // N_DEV
    n_half = n // 2

    def _lut(table, idx):
        out = jnp.int32(0)
        for j, v in enumerate(table):
            out = out + jnp.where(idx == j, jnp.int32(v), jnp.int32(0))
        return out

    def body(a_ref, b_ref, out_ref, cw_ref, ccw_ref,
             cw_send, cw_recv, ccw_send, ccw_recv):
        my = lax.axis_index("i")
        r = _lut(INV, my)
        cw_t = _lut(P, (r + 1) % N_DEV)
        ccw_t = _lut(P, (r + N_DEV - 1) % N_DEV)

        barrier_sem = pltpu.get_barrier_semaphore()
        for nbr in [cw_t, ccw_t]:
            pl.semaphore_signal(
                barrier_sem, inc=1,
                device_id=(nbr,), device_id_type=pl.DeviceIdType.MESH,
            )

        c0_cw = _lut(P, (r + N_DEV - 1) % N_DEV)
        c0_ccw = _lut(P, (r + 1) % N_DEV)
        cw_ref[0] = jnp.dot(
            a_ref[pl.ds(c0_cw * m_per, m_per), :], b_ref[:, :n_half],
            preferred_element_type=jnp.float32,
        )
        ccw_ref[0] = jnp.dot(
            a_ref[pl.ds(c0_ccw * m_per, m_per), :], b_ref[:, n_half:],
            preferred_element_type=jnp.float32,
        )

        pl.semaphore_wait(barrier_sem, 2)

        for s in range(N_DEV - 1):
            rdma_cw = pltpu.make_async_remote_copy(
                src_ref=cw_ref.at[s], dst_ref=cw_ref.at[s + 1],
                send_sem=cw_send.at[s], recv_sem=cw_recv.at[s + 1],
                device_id=(cw_t,), device_id_type=pl.DeviceIdType.MESH,
            )
            rdma_ccw = pltpu.make_async_remote_copy(
                src_ref=ccw_ref.at[s], dst_ref=ccw_ref.at[s + 1],
                send_sem=ccw_send.at[s], recv_sem=ccw_recv.at[s + 1],
                device_id=(ccw_t,), device_id_type=pl.DeviceIdType.MESH,
            )
            rdma_cw.start()
            rdma_ccw.start()

            c_cw = _lut(P, (r + N_DEV - 2 - s) % N_DEV)
            c_ccw = _lut(P, (r + 2 + s) % N_DEV)
            p_cw = jnp.dot(
                a_ref[pl.ds(c_cw * m_per, m_per), :], b_ref[:, :n_half],
                preferred_element_type=jnp.float32,
            )
            p_ccw = jnp.dot(
                a_ref[pl.ds(c_ccw * m_per, m_per), :], b_ref[:, n_half:],
                preferred_element_type=jnp.float32,
            )

            rdma_cw.wait()
            if s < N_DEV - 2:
                cw_ref[s + 1] = cw_ref[s + 1] + p_cw
            else:
                out_ref[:, :n_half] = cw_ref[s + 1] + p_cw
            rdma_ccw.wait()
            if s < N_DEV - 2:
                ccw_ref[s + 1] = ccw_ref[s + 1] + p_ccw
            else:
                out_ref[:, n_half:] = ccw_ref[s + 1] + p_ccw

    return pl.pallas_call(
        body,
        out_shape=jax.ShapeDtypeStruct((m_per, n), jnp.float32),
        in_specs=[
            pl.BlockSpec(memory_space=pltpu.VMEM),
            pl.BlockSpec(memory_space=pltpu.VMEM),
        ],
        out_specs=pl.BlockSpec(memory_space=pltpu.VMEM),
        scratch_shapes=[
            pltpu.VMEM((N_DEV, m_per, n_half), jnp.float32),
            pltpu.VMEM((N_DEV, m_per, n_half), jnp.float32),
            pltpu.SemaphoreType.DMA((N_DEV,)),
            pltpu.SemaphoreType.DMA((N_DEV,)),
            pltpu.SemaphoreType.DMA((N_DEV,)),
            pltpu.SemaphoreType.DMA((N_DEV,)),
        ],
        compiler_params=pltpu.CompilerParams(collective_id=0),
    )(A, B)


# device time: 64371 ns/iter; 2.0527x vs baseline; 1.3022x over previous
import jax
import jax.numpy as jnp
from jax import lax
from jax.experimental import pallas as pl
from jax.experimental.pallas import tpu as pltpu

N_DEV = 16
K_SUB = 2

P = [0, 1, 5, 4, 8, 9, 13, 12, 15, 14, 10, 11, 7, 6, 2, 3]
INV = [0] * N_DEV
for _r, _l in enumerate(P):
    INV[_l] = _r


def kernel(A, B):
    m, k_per = A.shape
    _, n = B.shape
    m_per = m // N_DEV
    n_half = n // 2
    w = n_half // K_SUB

    def _lut(table, idx):
        out = jnp.int32(0)
        for j, v in enumerate(table):
            out = out + jnp.where(idx == j, jnp.int32(v), jnp.int32(0))
        return out

    def body(a_ref, b_ref, out_ref, *scratch):
        slots = scratch[: 2 * K_SUB]
        send_sems = scratch[2 * K_SUB: 4 * K_SUB]
        recv_sems = scratch[4 * K_SUB: 6 * K_SUB]

        my = lax.axis_index("i")
        r = _lut(INV, my)
        cw_t = _lut(P, (r + 1) % N_DEV)
        ccw_t = _lut(P, (r + N_DEV - 1) % N_DEV)

        targets = [cw_t] * K_SUB + [ccw_t] * K_SUB

        def desc(q, s):
            return pltpu.make_async_remote_copy(
                src_ref=slots[q].at[s], dst_ref=slots[q].at[s + 1],
                send_sem=send_sems[q].at[s], recv_sem=recv_sems[q].at[s + 1],
                device_id=(targets[q],), device_id_type=pl.DeviceIdType.MESH,
            )

        barrier_sem = pltpu.get_barrier_semaphore()
        for nbr in [cw_t, ccw_t]:
            pl.semaphore_signal(
                barrier_sem, inc=1,
                device_id=(nbr,), device_id_type=pl.DeviceIdType.MESH,
            )

        c0_cw = _lut(P, (r + N_DEV - 1) % N_DEV)
        c0_ccw = _lut(P, (r + 1) % N_DEV)
        p0_cw = jnp.dot(
            a_ref[pl.ds(c0_cw * m_per, m_per), :], b_ref[:, :n_half],
            preferred_element_type=jnp.float32,
        )
        p0_ccw = jnp.dot(
            a_ref[pl.ds(c0_ccw * m_per, m_per), :], b_ref[:, n_half:],
            preferred_element_type=jnp.float32,
        )
        for q in range(2 * K_SUB):
            half = p0_cw if q < K_SUB else p0_ccw
            j = q % K_SUB
            slots[q][0] = half[:, j * w:(j + 1) * w]

        pl.semaphore_wait(barrier_sem, 2)

        for q in range(2 * K_SUB):
            desc(q, 0).start()

        for s in range(N_DEV - 1):
            c_cw = _lut(P, (r + N_DEV - 2 - s) % N_DEV)
            c_ccw = _lut(P, (r + 2 + s) % N_DEV)
            p_cw = jnp.dot(
                a_ref[pl.ds(c_cw * m_per, m_per), :], b_ref[:, :n_half],
                preferred_element_type=jnp.float32,
            )
            p_ccw = jnp.dot(
                a_ref[pl.ds(c_ccw * m_per, m_per), :], b_ref[:, n_half:],
                preferred_element_type=jnp.float32,
            )
            for q in range(2 * K_SUB):
                half = p_cw if q < K_SUB else p_ccw
                j = q % K_SUB
                part = half[:, j * w:(j + 1) * w]
                out_lo = (q % K_SUB) * w + (0 if q < K_SUB else n_half)
                desc(q, s).wait()
                if s < N_DEV - 2:
                    slots[q][s + 1] = slots[q][s + 1] + part
                    desc(q, s + 1).start()
                else:
                    out_ref[:, pl.ds(out_lo, w)] = slots[q][s + 1] + part

    slot_shape = pltpu.VMEM((N_DEV, m_per, w), jnp.float32)
    return pl.pallas_call(
        body,
        out_shape=jax.ShapeDtypeStruct((m_per, n), jnp.float32),
        in_specs=[
            pl.BlockSpec(memory_space=pltpu.VMEM),
            pl.BlockSpec(memory_space=pltpu.VMEM),
        ],
        out_specs=pl.BlockSpec(memory_space=pltpu.VMEM),
        scratch_shapes=(
            [slot_shape] * (2 * K_SUB)
            + [pltpu.SemaphoreType.DMA((N_DEV,))] * (2 * K_SUB)
            + [pltpu.SemaphoreType.DMA((N_DEV,))] * (2 * K_SUB)
        ),
        compiler_params=pltpu.CompilerParams(collective_id=0),
    )(A, B)


# device time: 63862 ns/iter; 2.0691x vs baseline; 1.0080x over previous
import jax
import jax.numpy as jnp
from jax import lax
from jax.experimental import pallas as pl
from jax.experimental.pallas import tpu as pltpu

N_DEV = 16
K_SUB = 3

P = [0, 1, 5, 4, 8, 9, 13, 12, 15, 14, 10, 11, 7, 6, 2, 3]
INV = [0] * N_DEV
for _r, _l in enumerate(P):
    INV[_l] = _r


def kernel(A, B):
    m, k_per = A.shape
    _, n = B.shape
    m_per = m // N_DEV
    n_half = n // 2
    w = n_half // K_SUB

    def _lut(table, idx):
        out = jnp.int32(0)
        for j, v in enumerate(table):
            out = out + jnp.where(idx == j, jnp.int32(v), jnp.int32(0))
        return out

    def body(a_ref, b_ref, out_ref, *scratch):
        slots = scratch[: 2 * K_SUB]
        send_sems = scratch[2 * K_SUB: 4 * K_SUB]
        recv_sems = scratch[4 * K_SUB: 6 * K_SUB]

        my = lax.axis_index("i")
        r = _lut(INV, my)
        cw_t = _lut(P, (r + 1) % N_DEV)
        ccw_t = _lut(P, (r + N_DEV - 1) % N_DEV)

        targets = [cw_t] * K_SUB + [ccw_t] * K_SUB

        def desc(q, s):
            return pltpu.make_async_remote_copy(
                src_ref=slots[q].at[s], dst_ref=slots[q].at[s + 1],
                send_sem=send_sems[q].at[s], recv_sem=recv_sems[q].at[s + 1],
                device_id=(targets[q],), device_id_type=pl.DeviceIdType.MESH,
            )

        barrier_sem = pltpu.get_barrier_semaphore()
        for nbr in [cw_t, ccw_t]:
            pl.semaphore_signal(
                barrier_sem, inc=1,
                device_id=(nbr,), device_id_type=pl.DeviceIdType.MESH,
            )

        c0_cw = _lut(P, (r + N_DEV - 1) % N_DEV)
        c0_ccw = _lut(P, (r + 1) % N_DEV)
        p0_cw = jnp.dot(
            a_ref[pl.ds(c0_cw * m_per, m_per), :], b_ref[:, :n_half],
            preferred_element_type=jnp.float32,
        )
        p0_ccw = jnp.dot(
            a_ref[pl.ds(c0_ccw * m_per, m_per), :], b_ref[:, n_half:],
            preferred_element_type=jnp.float32,
        )
        for q in range(2 * K_SUB):
            half = p0_cw if q < K_SUB else p0_ccw
            j = q % K_SUB
            slots[q][0] = half[:, j * w:(j + 1) * w]

        pl.semaphore_wait(barrier_sem, 2)

        for q in range(2 * K_SUB):
            desc(q, 0).start()

        for s in range(N_DEV - 1):
            c_cw = _lut(P, (r + N_DEV - 2 - s) % N_DEV)
            c_ccw = _lut(P, (r + 2 + s) % N_DEV)
            p_cw = jnp.dot(
                a_ref[pl.ds(c_cw * m_per, m_per), :], b_ref[:, :n_half],
                preferred_element_type=jnp.float32,
            )
            p_ccw = jnp.dot(
                a_ref[pl.ds(c_ccw * m_per, m_per), :], b_ref[:, n_half:],
                preferred_element_type=jnp.float32,
            )
            for q in range(2 * K_SUB):
                half = p_cw if q < K_SUB else p_ccw
                j = q % K_SUB
                part = half[:, j * w:(j + 1) * w]
                out_lo = (q % K_SUB) * w + (0 if q < K_SUB else n_half)
                desc(q, s).wait()
                if s < N_DEV - 2:
                    slots[q][s + 1] = slots[q][s + 1] + part
                    desc(q, s + 1).start()
                else:
                    out_ref[:, pl.ds(out_lo, w)] = slots[q][s + 1] + part

    slot_shape = pltpu.VMEM((N_DEV, m_per, w), jnp.float32)
    return pl.pallas_call(
        body,
        out_shape=jax.ShapeDtypeStruct((m_per, n), jnp.float32),
        in_specs=[
            pl.BlockSpec(memory_space=pltpu.VMEM),
            pl.BlockSpec(memory_space=pltpu.VMEM),
        ],
        out_specs=pl.BlockSpec(memory_space=pltpu.VMEM),
        scratch_shapes=(
            [slot_shape] * (2 * K_SUB)
            + [pltpu.SemaphoreType.DMA((N_DEV,))] * (2 * K_SUB)
            + [pltpu.SemaphoreType.DMA((N_DEV,))] * (2 * K_SUB)
        ),
        compiler_params=pltpu.CompilerParams(collective_id=0),
    )(A, B)


# device time: 61323 ns/iter; 2.1547x vs baseline; 1.0414x over previous
import jax
import jax.numpy as jnp
from jax import lax
from jax.experimental import pallas as pl
from jax.experimental.pallas import tpu as pltpu

N_DEV = 16
K_SUB = 2
CW_STEPS = 8
CCW_STEPS = 7

P = [0, 1, 5, 4, 8, 9, 13, 12, 15, 14, 10, 11, 7, 6, 2, 3]
INV = [0] * N_DEV
for _r, _l in enumerate(P):
    INV[_l] = _r


def kernel(A, B):
    m, k_per = A.shape
    _, n = B.shape
    m_per = m // N_DEV
    w = n // K_SUB

    def _lut(table, idx):
        out = jnp.int32(0)
        for j, v in enumerate(table):
            out = out + jnp.where(idx == j, jnp.int32(v), jnp.int32(0))
        return out

    def body(a_ref, b_ref, out_ref, *scratch):
        cw_slots = scratch[0:K_SUB]
        ccw_slots = scratch[K_SUB:2 * K_SUB]
        cw_send = scratch[2 * K_SUB:3 * K_SUB]
        cw_recv = scratch[3 * K_SUB:4 * K_SUB]
        ccw_send = scratch[4 * K_SUB:5 * K_SUB]
        ccw_recv = scratch[5 * K_SUB:6 * K_SUB]

        my = lax.axis_index("i")
        r = _lut(INV, my)
        cw_t = _lut(P, (r + 1) % N_DEV)
        ccw_t = _lut(P, (r + N_DEV - 1) % N_DEV)

        def desc_cw(q, s):
            return pltpu.make_async_remote_copy(
                src_ref=cw_slots[q].at[s], dst_ref=cw_slots[q].at[s + 1],
                send_sem=cw_send[q].at[s], recv_sem=cw_recv[q].at[s + 1],
                device_id=(cw_t,), device_id_type=pl.DeviceIdType.MESH,
            )

        def desc_ccw(q, s):
            return pltpu.make_async_remote_copy(
                src_ref=ccw_slots[q].at[s], dst_ref=ccw_slots[q].at[s + 1],
                send_sem=ccw_send[q].at[s], recv_sem=ccw_recv[q].at[s + 1],
                device_id=(ccw_t,), device_id_type=pl.DeviceIdType.MESH,
            )

        barrier_sem = pltpu.get_barrier_semaphore()
        for nbr in [cw_t, ccw_t]:
            pl.semaphore_signal(
                barrier_sem, inc=1,
                device_id=(nbr,), device_id_type=pl.DeviceIdType.MESH,
            )

        c0_cw = _lut(P, (r + 8) % N_DEV)
        c0_ccw = _lut(P, (r + 9) % N_DEV)
        p0_cw = jnp.dot(
            a_ref[pl.ds(c0_cw * m_per, m_per), :], b_ref[...],
            preferred_element_type=jnp.float32,
        )
        p0_ccw = jnp.dot(
            a_ref[pl.ds(c0_ccw * m_per, m_per), :], b_ref[...],
            preferred_element_type=jnp.float32,
        )
        for q in range(K_SUB):
            cw_slots[q][0] = p0_cw[:, q * w:(q + 1) * w]
            ccw_slots[q][0] = p0_ccw[:, q * w:(q + 1) * w]

        pl.semaphore_wait(barrier_sem, 2)

        for q in range(K_SUB):
            desc_cw(q, 0).start()
            desc_ccw(q, 0).start()

        for s in range(CW_STEPS):
            c_cw = _lut(P, (r + 7 - s) % N_DEV)
            p_cw = jnp.dot(
                a_ref[pl.ds(c_cw * m_per, m_per), :], b_ref[...],
                preferred_element_type=jnp.float32,
            )
            if s < CCW_STEPS:
                c_ccw = _lut(P, (r + 10 + s) % N_DEV)
                p_ccw = jnp.dot(
                    a_ref[pl.ds(c_ccw * m_per, m_per), :], b_ref[...],
                    preferred_element_type=jnp.float32,
                )
            for q in range(K_SUB):
                cols = slice(q * w, (q + 1) * w)
                desc_cw(q, s).wait()
                if s < CW_STEPS - 1:
                    cw_slots[q][s + 1] = cw_slots[q][s + 1] + p_cw[:, cols]
                    desc_cw(q, s + 1).start()
                else:
                    out_ref[:, pl.ds(q * w, w)] = (
                        cw_slots[q][CW_STEPS]
                        + ccw_slots[q][CCW_STEPS]
                        + p_cw[:, cols]
                    )
            if s < CCW_STEPS:
                for q in range(K_SUB):
                    cols = slice(q * w, (q + 1) * w)
                    desc_ccw(q, s).wait()
                    if s < CCW_STEPS - 1:
                        ccw_slots[q][s + 1] = (
                            ccw_slots[q][s + 1] + p_ccw[:, cols]
                        )
                        desc_ccw(q, s + 1).start()

    return pl.pallas_call(
        body,
        out_shape=jax.ShapeDtypeStruct((m_per, n), jnp.float32),
        in_specs=[
            pl.BlockSpec(memory_space=pltpu.VMEM),
            pl.BlockSpec(memory_space=pltpu.VMEM),
        ],
        out_specs=pl.BlockSpec(memory_space=pltpu.VMEM),
        scratch_shapes=(
            [pltpu.VMEM((CW_STEPS + 1, m_per, w), jnp.float32)] * K_SUB
            + [pltpu.VMEM((CCW_STEPS + 1, m_per, w), jnp.float32)] * K_SUB
            + [pltpu.SemaphoreType.DMA((CW_STEPS + 1,))] * K_SUB
            + [pltpu.SemaphoreType.DMA((CW_STEPS + 1,))] * K_SUB
            + [pltpu.SemaphoreType.DMA((CCW_STEPS + 1,))] * K_SUB
            + [pltpu.SemaphoreType.DMA((CCW_STEPS + 1,))] * K_SUB
        ),
        compiler_params=pltpu.CompilerParams(collective_id=0),
    )(A, B)
